# baseline (speedup 1.0000x reference)
import functools

import jax
import jax.numpy as jnp
from jax import lax
from jax.experimental import pallas as pl
from jax.experimental.pallas import tpu as pltpu
from jax.experimental.pallas import tpu_sc as plsc

BATCH = 1024
VOCAB = 100000
EMBED = 32

V_BLK = 2000
NV = VOCAB // V_BLK
G_BLK = V_BLK // 8

_NEG_INF = float("-inf")
_BIG_I32 = 2**31 - 1


def _argmax_body(xt_ref, out_ref, acc_v, acc_k):
    j = pl.program_id(0)

    @pl.when(j == 0)
    def _init():
        acc_v[...] = jnp.full((8, BATCH), _NEG_INF, jnp.float32)
        acc_k[...] = jnp.zeros((8, BATCH), jnp.int32)

    x3 = xt_ref[...].reshape(G_BLK, 8, BATCH)
    m = jnp.max(x3, axis=0)
    g = lax.broadcasted_iota(jnp.int32, (G_BLK, 8, BATCH), 0)
    kloc = jnp.min(jnp.where(x3 == m[None], g, _BIG_I32), axis=0)

    take = m > acc_v[...]
    acc_k[...] = jnp.where(take, kloc + j * G_BLK, acc_k[...])
    acc_v[...] = jnp.where(take, m, acc_v[...])

    @pl.when(j == NV - 1)
    def _done():
        av = acc_v[...]
        m8 = jnp.max(av, axis=0, keepdims=True)
        s = lax.broadcasted_iota(jnp.int32, (8, BATCH), 0)
        gidx = acc_k[...] * 8 + s
        cand = jnp.where(av == m8, gidx, _BIG_I32)
        out_ref[...] = jnp.min(cand, axis=0, keepdims=True)


def _row_argmax(xt):
    return pl.pallas_call(
        _argmax_body,
        grid=(NV,),
        in_specs=[pl.BlockSpec((V_BLK, BATCH), lambda j: (j, 0))],
        out_specs=pl.BlockSpec((1, BATCH), lambda j: (0, 0)),
        out_shape=jax.ShapeDtypeStruct((1, BATCH), jnp.int32),
        scratch_shapes=[
            pltpu.VMEM((8, BATCH), jnp.float32),
            pltpu.VMEM((8, BATCH), jnp.int32),
        ],
    )(xt)


F_STRIDE = 102400


def _sc_detile(table_t):
    info = plsc.get_sparse_core_info()
    nc, ns = info.num_cores, info.num_subcores

    mesh = plsc.VectorSubcoreMesh(core_axis_name="c", subcore_axis_name="s")

    @functools.partial(
        pl.kernel,
        mesh=mesh,
        out_type=jax.ShapeDtypeStruct((EMBED * F_STRIDE,), jnp.float32),
        scratch_types=[pltpu.VMEM((EMBED, 3200), jnp.float32)],
        compiler_params=pltpu.CompilerParams(
            use_tc_tiling_on_sc=True, needs_layout_passes=False
        ),
    )
    def detile(tt_hbm, out_hbm, vbuf):
        wid = lax.axis_index("s") * nc + lax.axis_index("c")
        q0 = wid * 24
        for i in range(24):
            src = tt_hbm.at[:, pl.ds(pl.multiple_of((q0 + i) * 128, 128), 128)]
            pltpu.sync_copy(src, vbuf.at[:, pl.ds(i * 128, 128)])

        @pl.when(wid < 13)
        def _extra():
            q = 768 + wid
            src = tt_hbm.at[:, pl.ds(pl.multiple_of(q * 128, 128), 128)]
            pltpu.sync_copy(src, vbuf.at[:, pl.ds(3072, 128)])

        for e in range(EMBED):
            dst = out_hbm.at[pl.ds(e * F_STRIDE + q0 * 128, 3072)]
            pltpu.sync_copy(vbuf.at[e, pl.ds(0, 3072)], dst)

        @pl.when(wid < 13)
        def _extra_out():
            for e in range(EMBED):
                start = e * F_STRIDE + (768 + wid) * 128
                pltpu.sync_copy(
                    vbuf.at[e, pl.ds(3072, 128)], out_hbm.at[pl.ds(start, 128)]
                )

    return detile(table_t)


def _sc_gather(table_t_flat, tail, idx):
    info = plsc.get_sparse_core_info()
    nc, ns = info.num_cores, info.num_subcores
    b_per_w = BATCH // (nc * ns)

    mesh = plsc.VectorSubcoreMesh(core_axis_name="c", subcore_axis_name="s")

    @functools.partial(
        pl.kernel,
        mesh=mesh,
        out_type=jax.ShapeDtypeStruct((BATCH, EMBED), jnp.float32),
        scratch_types=[
            pltpu.VMEM((b_per_w,), jnp.int32),
            pltpu.VMEM((8, 128), jnp.int32),
            pltpu.VMEM((8 * 128,), jnp.float32),
            pltpu.VMEM((EMBED * 32,), jnp.float32),
            pltpu.VMEM((b_per_w, EMBED), jnp.float32),
            pltpu.SemaphoreType.DMA,
        ],
        compiler_params=pltpu.CompilerParams(
            use_tc_tiling_on_sc=False, needs_layout_passes=False
        ),
    )
    def gather(flat_hbm, tail_hbm, idx_hbm, out_hbm, idx_v, gidx, colbuf,
               tail_v, rows_v, sem):
        wid = lax.axis_index("s") * nc + lax.axis_index("c")
        base = wid * b_per_w
        pltpu.sync_copy(idx_hbm.at[pl.ds(base, b_per_w)], idx_v)
        pltpu.sync_copy(tail_hbm, tail_v)
        h = [idx_v[pl.ds(0, 16)], idx_v[pl.ds(16, 16)]]
        for r in range(8):
            for l in range(8):
                e = r * 4 + l // 2
                gidx[r, pl.ds(l * 16, 16)] = h[l % 2] + e * F_STRIDE
        copies = [
            pltpu.async_copy(
                flat_hbm.at[gidx.at[r]], colbuf.at[pl.ds(128 * r, 128)], sem
            )
            for r in range(8)
        ]
        for c in copies:
            c.wait()
        for r in range(8):
            for l in range(8):
                e = r * 4 + l // 2
                hh = h[l % 2]
                msk = hh >= (VOCAB - 32)
                tv = plsc.load_gather(
                    tail_v, [hh - (VOCAB - 32) + e * 32], mask=msk
                )
                cur = colbuf[pl.ds(r * 128 + l * 16, 16)]
                colbuf[pl.ds(r * 128 + l * 16, 16)] = jnp.where(msk, tv, cur)
        lane = lax.broadcasted_iota(jnp.int32, (16,), 0)
        for b in range(b_per_w):
            for eh in range(2):
                p = (lane + 16 * eh) * EMBED + b
                rows_v[b, pl.ds(16 * eh, 16)] = plsc.load_gather(colbuf, [p])
        pltpu.sync_copy(rows_v, out_hbm.at[pl.ds(base, b_per_w)])

    return gather(table_t_flat, tail, idx)


def kernel(x, table):
    tt = table.T
    flat = _sc_detile(tt)
    tail = tt[:, VOCAB - 32:].reshape(-1)
    idx = _row_argmax(x.T).reshape(BATCH)
    return _sc_gather(flat, tail, idx)


# speedup vs baseline: 1.0342x; 1.0342x over previous
import functools

import jax
import jax.numpy as jnp
from jax import lax
from jax.experimental import pallas as pl
from jax.experimental.pallas import tpu as pltpu
from jax.experimental.pallas import tpu_sc as plsc

BATCH = 1024
VOCAB = 100000
EMBED = 32

V_BLK = 5000
NV = VOCAB // V_BLK
G_BLK = V_BLK // 8

_NEG_INF = float("-inf")
_BIG_I32 = 2**31 - 1


def _argmax_body(xt_ref, out_ref, acc_v, acc_k):
    j = pl.program_id(0)

    @pl.when(j == 0)
    def _init():
        acc_v[...] = jnp.full((8, BATCH), _NEG_INF, jnp.float32)
        acc_k[...] = jnp.zeros((8, BATCH), jnp.int32)

    x3 = xt_ref[...].reshape(G_BLK, 8, BATCH)
    m = jnp.max(x3, axis=0)
    g = lax.broadcasted_iota(jnp.int32, (G_BLK, 8, BATCH), 0)
    kloc = jnp.min(jnp.where(x3 == m[None], g, _BIG_I32), axis=0)

    take = m > acc_v[...]
    acc_k[...] = jnp.where(take, kloc + j * G_BLK, acc_k[...])
    acc_v[...] = jnp.where(take, m, acc_v[...])

    @pl.when(j == NV - 1)
    def _done():
        av = acc_v[...]
        m8 = jnp.max(av, axis=0, keepdims=True)
        s = lax.broadcasted_iota(jnp.int32, (8, BATCH), 0)
        gidx = acc_k[...] * 8 + s
        cand = jnp.where(av == m8, gidx, _BIG_I32)
        out_ref[...] = jnp.min(cand, axis=0, keepdims=True)


def _row_argmax(xt):
    return pl.pallas_call(
        _argmax_body,
        grid=(NV,),
        in_specs=[pl.BlockSpec((V_BLK, BATCH), lambda j: (j, 0))],
        out_specs=pl.BlockSpec((1, BATCH), lambda j: (0, 0)),
        out_shape=jax.ShapeDtypeStruct((1, BATCH), jnp.int32),
        scratch_shapes=[
            pltpu.VMEM((8, BATCH), jnp.float32),
            pltpu.VMEM((8, BATCH), jnp.int32),
        ],
    )(xt)


F_STRIDE = 102400


def _sc_detile(table_t):
    info = plsc.get_sparse_core_info()
    nc, ns = info.num_cores, info.num_subcores

    mesh = plsc.VectorSubcoreMesh(core_axis_name="c", subcore_axis_name="s")

    @functools.partial(
        pl.kernel,
        mesh=mesh,
        out_type=jax.ShapeDtypeStruct((EMBED * F_STRIDE,), jnp.float32),
        scratch_types=[pltpu.VMEM((EMBED, 3200), jnp.float32)],
        compiler_params=pltpu.CompilerParams(
            use_tc_tiling_on_sc=True, needs_layout_passes=False
        ),
    )
    def detile(tt_hbm, out_hbm, vbuf):
        wid = lax.axis_index("s") * nc + lax.axis_index("c")
        q0 = wid * 24
        for i in range(24):
            src = tt_hbm.at[:, pl.ds(pl.multiple_of((q0 + i) * 128, 128), 128)]
            pltpu.sync_copy(src, vbuf.at[:, pl.ds(i * 128, 128)])

        @pl.when(wid < 13)
        def _extra():
            q = 768 + wid
            src = tt_hbm.at[:, pl.ds(pl.multiple_of(q * 128, 128), 128)]
            pltpu.sync_copy(src, vbuf.at[:, pl.ds(3072, 128)])

        for e in range(EMBED):
            dst = out_hbm.at[pl.ds(e * F_STRIDE + q0 * 128, 3072)]
            pltpu.sync_copy(vbuf.at[e, pl.ds(0, 3072)], dst)

        @pl.when(wid < 13)
        def _extra_out():
            for e in range(EMBED):
                start = e * F_STRIDE + (768 + wid) * 128
                pltpu.sync_copy(
                    vbuf.at[e, pl.ds(3072, 128)], out_hbm.at[pl.ds(start, 128)]
                )

    return detile(table_t)


def _sc_gather(table_t_flat, tail, idx):
    info = plsc.get_sparse_core_info()
    nc, ns = info.num_cores, info.num_subcores
    b_per_w = BATCH // (nc * ns)

    mesh = plsc.VectorSubcoreMesh(core_axis_name="c", subcore_axis_name="s")

    @functools.partial(
        pl.kernel,
        mesh=mesh,
        out_type=jax.ShapeDtypeStruct((BATCH, EMBED), jnp.float32),
        scratch_types=[
            pltpu.VMEM((b_per_w,), jnp.int32),
            pltpu.VMEM((8, 128), jnp.int32),
            pltpu.VMEM((8 * 128,), jnp.float32),
            pltpu.VMEM((EMBED * 32,), jnp.float32),
            pltpu.VMEM((b_per_w, EMBED), jnp.float32),
            pltpu.SemaphoreType.DMA,
        ],
        compiler_params=pltpu.CompilerParams(
            use_tc_tiling_on_sc=False, needs_layout_passes=False
        ),
    )
    def gather(flat_hbm, tail_hbm, idx_hbm, out_hbm, idx_v, gidx, colbuf,
               tail_v, rows_v, sem):
        wid = lax.axis_index("s") * nc + lax.axis_index("c")
        base = wid * b_per_w
        pltpu.sync_copy(idx_hbm.at[pl.ds(base, b_per_w)], idx_v)
        pltpu.sync_copy(tail_hbm, tail_v)
        h = [idx_v[pl.ds(0, 16)], idx_v[pl.ds(16, 16)]]
        for r in range(8):
            for l in range(8):
                e = r * 4 + l // 2
                gidx[r, pl.ds(l * 16, 16)] = h[l % 2] + e * F_STRIDE
        copies = [
            pltpu.async_copy(
                flat_hbm.at[gidx.at[r]], colbuf.at[pl.ds(128 * r, 128)], sem
            )
            for r in range(8)
        ]
        for c in copies:
            c.wait()
        for r in range(8):
            for l in range(8):
                e = r * 4 + l // 2
                hh = h[l % 2]
                msk = hh >= (VOCAB - 32)
                tv = plsc.load_gather(
                    tail_v, [hh - (VOCAB - 32) + e * 32], mask=msk
                )
                cur = colbuf[pl.ds(r * 128 + l * 16, 16)]
                colbuf[pl.ds(r * 128 + l * 16, 16)] = jnp.where(msk, tv, cur)
        lane = lax.broadcasted_iota(jnp.int32, (16,), 0)
        for b in range(b_per_w):
            for eh in range(2):
                p = (lane + 16 * eh) * EMBED + b
                rows_v[b, pl.ds(16 * eh, 16)] = plsc.load_gather(colbuf, [p])
        pltpu.sync_copy(rows_v, out_hbm.at[pl.ds(base, b_per_w)])

    return gather(table_t_flat, tail, idx)


def kernel(x, table):
    tt = table.T
    flat = _sc_detile(tt)
    tail = tt[:, VOCAB - 32:].reshape(-1)
    idx = _row_argmax(x.T).reshape(BATCH)
    return _sc_gather(flat, tail, idx)


# speedup vs baseline: 1.0661x; 1.0309x over previous
import functools

import jax
import jax.numpy as jnp
from jax import lax
from jax.experimental import pallas as pl
from jax.experimental.pallas import tpu as pltpu
from jax.experimental.pallas import tpu_sc as plsc

BATCH = 1024
VOCAB = 100000
EMBED = 32

V_BLK = 4000
NV = VOCAB // V_BLK
G_BLK = V_BLK // 8

_NEG_INF = float("-inf")
_BIG_I32 = 2**31 - 1


def _argmax_body(xt_ref, out_ref, acc_v, acc_k):
    j = pl.program_id(0)

    @pl.when(j == 0)
    def _init():
        acc_v[...] = jnp.full((8, BATCH), _NEG_INF, jnp.float32)
        acc_k[...] = jnp.zeros((8, BATCH), jnp.int32)

    x3 = xt_ref[...].reshape(G_BLK, 8, BATCH)
    m = jnp.max(x3, axis=0)
    g = lax.broadcasted_iota(jnp.int32, (G_BLK, 8, BATCH), 0)
    kloc = jnp.min(jnp.where(x3 == m[None], g, _BIG_I32), axis=0)

    take = m > acc_v[...]
    acc_k[...] = jnp.where(take, kloc + j * G_BLK, acc_k[...])
    acc_v[...] = jnp.where(take, m, acc_v[...])

    @pl.when(j == NV - 1)
    def _done():
        av = acc_v[...]
        m8 = jnp.max(av, axis=0, keepdims=True)
        s = lax.broadcasted_iota(jnp.int32, (8, BATCH), 0)
        gidx = acc_k[...] * 8 + s
        cand = jnp.where(av == m8, gidx, _BIG_I32)
        out_ref[...] = jnp.min(cand, axis=0, keepdims=True)


def _row_argmax(xt):
    return pl.pallas_call(
        _argmax_body,
        grid=(NV,),
        in_specs=[pl.BlockSpec((V_BLK, BATCH), lambda j: (j, 0))],
        out_specs=pl.BlockSpec((1, BATCH), lambda j: (0, 0)),
        out_shape=jax.ShapeDtypeStruct((1, BATCH), jnp.int32),
        scratch_shapes=[
            pltpu.VMEM((8, BATCH), jnp.float32),
            pltpu.VMEM((8, BATCH), jnp.int32),
        ],
    )(xt)


F_STRIDE = 102400


def _sc_detile(table_t):
    info = plsc.get_sparse_core_info()
    nc, ns = info.num_cores, info.num_subcores

    mesh = plsc.VectorSubcoreMesh(core_axis_name="c", subcore_axis_name="s")

    @functools.partial(
        pl.kernel,
        mesh=mesh,
        out_type=jax.ShapeDtypeStruct((EMBED * F_STRIDE,), jnp.float32),
        scratch_types=[pltpu.VMEM((EMBED, 3200), jnp.float32)],
        compiler_params=pltpu.CompilerParams(
            use_tc_tiling_on_sc=True, needs_layout_passes=False
        ),
    )
    def detile(tt_hbm, out_hbm, vbuf):
        wid = lax.axis_index("s") * nc + lax.axis_index("c")
        q0 = wid * 24
        for i in range(24):
            src = tt_hbm.at[:, pl.ds(pl.multiple_of((q0 + i) * 128, 128), 128)]
            pltpu.sync_copy(src, vbuf.at[:, pl.ds(i * 128, 128)])

        @pl.when(wid < 13)
        def _extra():
            q = 768 + wid
            src = tt_hbm.at[:, pl.ds(pl.multiple_of(q * 128, 128), 128)]
            pltpu.sync_copy(src, vbuf.at[:, pl.ds(3072, 128)])

        for e in range(EMBED):
            dst = out_hbm.at[pl.ds(e * F_STRIDE + q0 * 128, 3072)]
            pltpu.sync_copy(vbuf.at[e, pl.ds(0, 3072)], dst)

        @pl.when(wid < 13)
        def _extra_out():
            for e in range(EMBED):
                start = e * F_STRIDE + (768 + wid) * 128
                pltpu.sync_copy(
                    vbuf.at[e, pl.ds(3072, 128)], out_hbm.at[pl.ds(start, 128)]
                )

    return detile(table_t)


def _sc_gather(table_t_flat, tail, idx):
    info = plsc.get_sparse_core_info()
    nc, ns = info.num_cores, info.num_subcores
    b_per_w = BATCH // (nc * ns)

    mesh = plsc.VectorSubcoreMesh(core_axis_name="c", subcore_axis_name="s")

    @functools.partial(
        pl.kernel,
        mesh=mesh,
        out_type=jax.ShapeDtypeStruct((BATCH, EMBED), jnp.float32),
        scratch_types=[
            pltpu.VMEM((b_per_w,), jnp.int32),
            pltpu.VMEM((8, 128), jnp.int32),
            pltpu.VMEM((8 * 128,), jnp.float32),
            pltpu.VMEM((EMBED * 32,), jnp.float32),
            pltpu.VMEM((b_per_w, EMBED), jnp.float32),
            pltpu.SemaphoreType.DMA,
        ],
        compiler_params=pltpu.CompilerParams(
            use_tc_tiling_on_sc=False, needs_layout_passes=False
        ),
    )
    def gather(flat_hbm, tail_hbm, idx_hbm, out_hbm, idx_v, gidx, colbuf,
               tail_v, rows_v, sem):
        wid = lax.axis_index("s") * nc + lax.axis_index("c")
        base = wid * b_per_w
        pltpu.sync_copy(idx_hbm.at[pl.ds(base, b_per_w)], idx_v)
        h = [idx_v[pl.ds(0, 16)], idx_v[pl.ds(16, 16)]]
        for r in range(8):
            for l in range(8):
                e = r * 4 + l // 2
                gidx[r, pl.ds(l * 16, 16)] = h[l % 2] + e * F_STRIDE
        copies = [
            pltpu.async_copy(
                flat_hbm.at[gidx.at[r]], colbuf.at[pl.ds(128 * r, 128)], sem
            )
            for r in range(8)
        ]
        for c in copies:
            c.wait()
        hmax = jnp.maximum(jnp.max(h[0], axis=0), jnp.max(h[1], axis=0))

        @pl.when(hmax >= VOCAB - 32)
        def _tail_patch():
            pltpu.sync_copy(tail_hbm, tail_v)
            for r in range(8):
                for l in range(8):
                    e = r * 4 + l // 2
                    hh = h[l % 2]
                    msk = hh >= (VOCAB - 32)
                    tv = plsc.load_gather(
                        tail_v, [hh - (VOCAB - 32) + e * 32], mask=msk
                    )
                    cur = colbuf[pl.ds(r * 128 + l * 16, 16)]
                    colbuf[pl.ds(r * 128 + l * 16, 16)] = jnp.where(msk, tv, cur)
        lane = lax.broadcasted_iota(jnp.int32, (16,), 0)
        for b in range(b_per_w):
            for eh in range(2):
                p = (lane + 16 * eh) * EMBED + b
                rows_v[b, pl.ds(16 * eh, 16)] = plsc.load_gather(colbuf, [p])
        pltpu.sync_copy(rows_v, out_hbm.at[pl.ds(base, b_per_w)])

    return gather(table_t_flat, tail, idx)


def kernel(x, table):
    tt = table.T
    flat = _sc_detile(tt)
    tail = tt[:, VOCAB - 32:].reshape(-1)
    idx = _row_argmax(x.T).reshape(BATCH)
    return _sc_gather(flat, tail, idx)
